# baseline (speedup 1.0000x reference)
import functools

import jax
import jax.numpy as jnp
from jax import lax
from jax.experimental import pallas as pl
from jax.experimental.pallas import tpu as pltpu
from jax.experimental.pallas import tpu_sc as plsc

NUM_EMBEDDINGS = 8192
EMBEDDING_DIM = 32
BETA = 0.25


_ROW_BLOCK = 256


def _rowsum32(x):
    acc = ((x[:, 0:8] + x[:, 8:16]) + x[:, 16:24]) + x[:, 24:32]
    acc = acc[:, 0:4] + acc[:, 4:8]
    acc = acc[:, 0:2] + acc[:, 2:4]
    return acc[:, 0:1] + acc[:, 1:2]


def _colsum32(x):
    acc = ((x[0:8, :] + x[8:16, :]) + x[16:24, :]) + x[24:32, :]
    acc = acc[0:4, :] + acc[4:8, :]
    acc = acc[0:2, :] + acc[2:4, :]
    return acc[0:1, :] + acc[1:2, :]


def _seqsum32(x):
    acc = x[:, 0:1]
    for i in range(1, 32):
        acc = acc + x[:, i:i + 1]
    return acc


def _argmin_body(flat_ref, cbt_ref, idx_ref):
    a = flat_ref[...]
    cbt = cbt_ref[...]
    a2 = _seqsum32(a * a)
    b2 = _colsum32(cbt * cbt)
    dot = lax.dot_general(a, cbt,
                          (((1,), (0,)), ((), ())),
                          preferred_element_type=jnp.float32)
    d2 = a2 + b2 - 2.0 * dot
    dist = jnp.sqrt(jnp.maximum(d2, 0.0))
    m = jnp.min(dist, axis=1, keepdims=True)
    col = lax.broadcasted_iota(jnp.int32, dist.shape, 1)
    big = jnp.int32(2**31 - 1)
    idx_ref[...] = jnp.min(jnp.where(dist == m, col, big), axis=1)


def _matched_indices(flat, codebook_t):
    n, c = flat.shape
    e = codebook_t.shape[1]
    grid = n // _ROW_BLOCK
    return pl.pallas_call(
        _argmin_body,
        grid=(grid,),
        in_specs=[
            pl.BlockSpec((_ROW_BLOCK, c), lambda i: (i, 0)),
            pl.BlockSpec((c, e), lambda i: (0, 0)),
        ],
        out_specs=pl.BlockSpec((_ROW_BLOCK,), lambda i: (i,)),
        out_shape=jax.ShapeDtypeStruct((n,), jnp.int32),
    )(flat, codebook_t)


_SC_NUM_CORES = 2
_SC_NUM_SUBCORES = 16
_SC_WORKERS = _SC_NUM_CORES * _SC_NUM_SUBCORES


def _make_sc_gather(n, e, d):
    b_per_w = n // _SC_WORKERS
    mesh = plsc.VectorSubcoreMesh(core_axis_name="c", subcore_axis_name="s")

    @functools.partial(
        pl.kernel,
        mesh=mesh,
        out_type=jax.ShapeDtypeStruct((n, d), jnp.float32),
        scratch_types=[
            pltpu.VMEM((b_per_w,), jnp.int32),
            pltpu.VMEM((b_per_w, d), jnp.float32),
            pltpu.SemaphoreType.DMA,
        ],
    )
    def gather(table_hbm, idx_hbm, out_hbm, idx_v, rows_v, sem):
        wid = lax.axis_index("s") * _SC_NUM_CORES + lax.axis_index("c")
        base = wid * b_per_w
        pltpu.sync_copy(idx_hbm.at[pl.ds(base, b_per_w)], idx_v)
        pltpu.async_copy(table_hbm.at[idx_v], rows_v, sem).wait()
        pltpu.sync_copy(rows_v, out_hbm.at[pl.ds(base, b_per_w)])

    return gather


def _st_loss_body(flat_ref, q_ref, qst_ref, ssum_ref):
    f = flat_ref[...]
    q = q_ref[...]
    diff = q - f
    qst_ref[...] = f + diff
    ssum_ref[...] = jnp.sum(diff * diff)[None, None]


def _st_and_loss(flat, quant):
    n, c = flat.shape
    return pl.pallas_call(
        _st_loss_body,
        out_shape=(
            jax.ShapeDtypeStruct((n, c), jnp.float32),
            jax.ShapeDtypeStruct((1, 1), jnp.float32),
        ),
    )(flat, quant)


def kernel(inputs, codebook):
    b, c, h, w = inputs.shape
    n = b * h * w
    flat = jnp.transpose(inputs, (0, 2, 3, 1)).reshape((n, c))
    matched = _matched_indices(flat, codebook.T)
    table = jnp.pad(codebook, ((0, 0), (0, 128 - c)))
    quant = _make_sc_gather(n, codebook.shape[0], 128)(table, matched)[:, :c]
    qst, ssum = _st_and_loss(flat, quant)
    q_loss = ssum[0, 0] / jnp.float32(n * c)
    e_loss = q_loss
    vq_loss = q_loss + BETA * e_loss
    quantized_st = qst.reshape((b, h, w, c)).transpose((0, 3, 1, 2))
    return (quantized_st, q_loss, e_loss, vq_loss,
            matched.reshape((b, h, w)))


# speedup vs baseline: 1.0052x; 1.0052x over previous
import functools

import jax
import jax.numpy as jnp
from jax import lax
from jax.experimental import pallas as pl
from jax.experimental.pallas import tpu as pltpu
from jax.experimental.pallas import tpu_sc as plsc

NUM_EMBEDDINGS = 8192
EMBEDDING_DIM = 32
BETA = 0.25


_ROW_BLOCK = 1024


def _rowsum32(x):
    acc = ((x[:, 0:8] + x[:, 8:16]) + x[:, 16:24]) + x[:, 24:32]
    acc = acc[:, 0:4] + acc[:, 4:8]
    acc = acc[:, 0:2] + acc[:, 2:4]
    return acc[:, 0:1] + acc[:, 1:2]


def _colsum32(x):
    acc = ((x[0:8, :] + x[8:16, :]) + x[16:24, :]) + x[24:32, :]
    acc = acc[0:4, :] + acc[4:8, :]
    acc = acc[0:2, :] + acc[2:4, :]
    return acc[0:1, :] + acc[1:2, :]


def _seqsum32(x):
    acc = x[:, 0:1]
    for i in range(1, 32):
        acc = acc + x[:, i:i + 1]
    return acc


def _argmin_body(flat_ref, cbt_ref, idx_ref):
    a = flat_ref[...]
    cbt = cbt_ref[...]
    a2 = _seqsum32(a * a)
    b2 = _colsum32(cbt * cbt)
    dot = lax.dot_general(a, cbt,
                          (((1,), (0,)), ((), ())),
                          preferred_element_type=jnp.float32)
    d2 = a2 + b2 - 2.0 * dot
    dist = jnp.sqrt(jnp.maximum(d2, 0.0))
    m = jnp.min(dist, axis=1, keepdims=True)
    col = lax.broadcasted_iota(jnp.int32, dist.shape, 1)
    big = jnp.int32(2**31 - 1)
    idx_ref[...] = jnp.min(jnp.where(dist == m, col, big), axis=1)


def _matched_indices(flat, codebook_t):
    n, c = flat.shape
    e = codebook_t.shape[1]
    grid = n // _ROW_BLOCK
    return pl.pallas_call(
        _argmin_body,
        grid=(grid,),
        in_specs=[
            pl.BlockSpec((_ROW_BLOCK, c), lambda i: (i, 0)),
            pl.BlockSpec((c, e), lambda i: (0, 0)),
        ],
        out_specs=pl.BlockSpec((_ROW_BLOCK,), lambda i: (i,)),
        out_shape=jax.ShapeDtypeStruct((n,), jnp.int32),
    )(flat, codebook_t)


_SC_NUM_CORES = 2
_SC_NUM_SUBCORES = 16
_SC_WORKERS = _SC_NUM_CORES * _SC_NUM_SUBCORES


def _make_sc_gather(n, e, d):
    b_per_w = n // _SC_WORKERS
    mesh = plsc.VectorSubcoreMesh(core_axis_name="c", subcore_axis_name="s")

    @functools.partial(
        pl.kernel,
        mesh=mesh,
        out_type=jax.ShapeDtypeStruct((n, d), jnp.float32),
        scratch_types=[
            pltpu.VMEM((b_per_w,), jnp.int32),
            pltpu.VMEM((b_per_w, d), jnp.float32),
            pltpu.SemaphoreType.DMA,
        ],
    )
    def gather(table_hbm, idx_hbm, out_hbm, idx_v, rows_v, sem):
        wid = lax.axis_index("s") * _SC_NUM_CORES + lax.axis_index("c")
        base = wid * b_per_w
        pltpu.sync_copy(idx_hbm.at[pl.ds(base, b_per_w)], idx_v)
        pltpu.async_copy(table_hbm.at[idx_v], rows_v, sem).wait()
        pltpu.sync_copy(rows_v, out_hbm.at[pl.ds(base, b_per_w)])

    return gather


def _st_loss_body(flat_ref, q_ref, qst_ref, ssum_ref):
    f = flat_ref[...]
    q = q_ref[...]
    diff = q - f
    qst_ref[...] = f + diff
    ssum_ref[...] = jnp.sum(diff * diff)[None, None]


def _st_and_loss(flat, quant):
    n, c = flat.shape
    return pl.pallas_call(
        _st_loss_body,
        out_shape=(
            jax.ShapeDtypeStruct((n, c), jnp.float32),
            jax.ShapeDtypeStruct((1, 1), jnp.float32),
        ),
    )(flat, quant)


def kernel(inputs, codebook):
    b, c, h, w = inputs.shape
    n = b * h * w
    flat = jnp.transpose(inputs, (0, 2, 3, 1)).reshape((n, c))
    matched = _matched_indices(flat, codebook.T)
    table = jnp.pad(codebook, ((0, 0), (0, 128 - c)))
    quant = _make_sc_gather(n, codebook.shape[0], 128)(table, matched)[:, :c]
    qst, ssum = _st_and_loss(flat, quant)
    q_loss = ssum[0, 0] / jnp.float32(n * c)
    e_loss = q_loss
    vq_loss = q_loss + BETA * e_loss
    quantized_st = qst.reshape((b, h, w, c)).transpose((0, 3, 1, 2))
    return (quantized_st, q_loss, e_loss, vq_loss,
            matched.reshape((b, h, w)))


# speedup vs baseline: 1.0057x; 1.0005x over previous
import functools

import jax
import jax.numpy as jnp
from jax import lax
from jax.experimental import pallas as pl
from jax.experimental.pallas import tpu as pltpu
from jax.experimental.pallas import tpu_sc as plsc

NUM_EMBEDDINGS = 8192
EMBEDDING_DIM = 32
BETA = 0.25


_ROW_BLOCK = 1024


def _colsum32(x):
    acc = ((x[0:8, :] + x[8:16, :]) + x[16:24, :]) + x[24:32, :]
    acc = acc[0:4, :] + acc[4:8, :]
    acc = acc[0:2, :] + acc[2:4, :]
    return acc[0:1, :] + acc[1:2, :]


def _seqsum32(x):
    acc = x[:, 0:1]
    for i in range(1, 32):
        acc = acc + x[:, i:i + 1]
    return acc


def _argmin_body(flat_ref, cbt_ref, idx_ref):
    a = flat_ref[...]
    cbt = cbt_ref[...]
    a2 = _seqsum32(a * a)
    b2 = _colsum32(cbt * cbt)
    dot = lax.dot_general(a, cbt,
                          (((1,), (0,)), ((), ())),
                          preferred_element_type=jnp.float32)
    d2 = a2 + b2 - 2.0 * dot
    dist = jnp.sqrt(jnp.maximum(d2, 0.0))
    m = jnp.min(dist, axis=1, keepdims=True)
    col = lax.broadcasted_iota(jnp.int32, dist.shape, 1)
    big = jnp.int32(2**31 - 1)
    idx_ref[...] = jnp.min(jnp.where(dist == m, col, big), axis=1)


def _matched_indices(flat, codebook_t):
    n, c = flat.shape
    e = codebook_t.shape[1]
    grid = n // _ROW_BLOCK
    return pl.pallas_call(
        _argmin_body,
        grid=(grid,),
        in_specs=[
            pl.BlockSpec((_ROW_BLOCK, c), lambda i: (i, 0)),
            pl.BlockSpec((c, e), lambda i: (0, 0)),
        ],
        out_specs=pl.BlockSpec((_ROW_BLOCK,), lambda i: (i,)),
        out_shape=jax.ShapeDtypeStruct((n,), jnp.int32),
    )(flat, codebook_t)


_SC_NUM_CORES = 2
_SC_NUM_SUBCORES = 16
_SC_WORKERS = _SC_NUM_CORES * _SC_NUM_SUBCORES


def _make_sc_gather(n, e, d, dtype):
    b_per_w = n // _SC_WORKERS
    mesh = plsc.VectorSubcoreMesh(core_axis_name="c", subcore_axis_name="s")

    @functools.partial(
        pl.kernel,
        mesh=mesh,
        out_type=jax.ShapeDtypeStruct((n, 128), dtype),
        scratch_types=[
            pltpu.VMEM((b_per_w,), jnp.int32),
            pltpu.VMEM((b_per_w, 128), dtype),
            pltpu.SemaphoreType.DMA,
        ],
    )
    def gather(table_hbm, idx_hbm, out_hbm, idx_v, rows_v, sem):
        wid = lax.axis_index("s") * _SC_NUM_CORES + lax.axis_index("c")
        base = wid * b_per_w
        pltpu.sync_copy(idx_hbm.at[pl.ds(base, b_per_w)], idx_v)
        pltpu.async_copy(table_hbm.at[idx_v], rows_v, sem).wait()
        pltpu.sync_copy(rows_v, out_hbm.at[pl.ds(base, b_per_w)])

    return gather


def _st_loss_body(flat_ref, q_ref, qst_ref, ssum_ref):
    f = flat_ref[...]
    q = q_ref[...]
    diff = q - f
    qst_ref[...] = f + diff
    ssum_ref[...] = jnp.sum(diff * diff)[None, None]


def _st_and_loss(flat, quant):
    n, c = flat.shape
    return pl.pallas_call(
        _st_loss_body,
        out_shape=(
            jax.ShapeDtypeStruct((n, c), jnp.float32),
            jax.ShapeDtypeStruct((1, 1), jnp.float32),
        ),
    )(flat, quant)


def kernel(inputs, codebook):
    b, c, h, w = inputs.shape
    n = b * h * w
    flat = jnp.transpose(inputs, (0, 2, 3, 1)).reshape((n, c))
    matched = _matched_indices(flat, codebook.T)
    table = jnp.pad(codebook, ((0, 0), (0, 128 - c)))
    gathered = _make_sc_gather(n, codebook.shape[0], c, jnp.float32)(table, matched)
    quant = gathered[:, :c]
    qst, ssum = _st_and_loss(flat, quant)
    q_loss = ssum[0, 0] / jnp.float32(n * c)
    e_loss = q_loss
    vq_loss = q_loss + BETA * e_loss
    quantized_st = qst.reshape((b, h, w, c)).transpose((0, 3, 1, 2))
    return (quantized_st, q_loss, e_loss, vq_loss,
            matched.reshape((b, h, w)))
